# baseline (speedup 1.0000x reference)
import functools

import jax
import jax.numpy as jnp
from jax import lax
from jax.experimental import pallas as pl
from jax.experimental.pallas import tpu as pltpu
from jax.experimental.pallas import tpu_sc as plsc

S = 50
NB = 16384
DIM = 64
W = 32
CHUNK = 128
ROWS = NB * S
RPW = ROWS // W
NCH = RPW // CHUNK
NBUF = 4
INFLIGHT = 3


def _embedding_lookup(xg, table):
    mesh = plsc.VectorSubcoreMesh(core_axis_name="c", subcore_axis_name="s")

    @functools.partial(
        pl.kernel,
        out_type=jax.ShapeDtypeStruct((ROWS // CHUNK, CHUNK, DIM),
                                      jnp.float32),
        mesh=mesh,
        scratch_types=[
            pltpu.VMEM((NCH, CHUNK), jnp.int32),
            pltpu.VMEM((NBUF, CHUNK, DIM), jnp.float32),
            pltpu.SemaphoreType.DMA((NBUF,)),
            pltpu.SemaphoreType.DMA((NBUF,)),
        ],
        compiler_params=pltpu.CompilerParams(use_tc_tiling_on_sc=False),
    )
    def body(xg_hbm, table_hbm, out_hbm, idx_v, rows_v, g_sem, o_sem):
        wid = lax.axis_index("s") * 2 + lax.axis_index("c")
        ch_base = wid * NCH
        pltpu.sync_copy(xg_hbm.at[wid], idx_v)

        def start_gather(j):
            buf = lax.rem(j, NBUF)
            pltpu.async_copy(table_hbm.at[idx_v.at[j]], rows_v.at[buf],
                             g_sem.at[buf])

        def wait_gather(j):
            buf = lax.rem(j, NBUF)
            pltpu.make_async_copy(table_hbm.at[idx_v.at[j]], rows_v.at[buf],
                                  g_sem.at[buf]).wait()

        def start_out(j):
            buf = lax.rem(j, NBUF)
            pltpu.async_copy(rows_v.at[buf], out_hbm.at[ch_base + j],
                             o_sem.at[buf])

        def wait_out(j):
            buf = lax.rem(j, NBUF)
            pltpu.make_async_copy(rows_v.at[buf], out_hbm.at[ch_base + j],
                                  o_sem.at[buf]).wait()

        for j in range(INFLIGHT):
            start_gather(j)

        def step(j, carry):
            wait_gather(j)
            start_out(j)

            @pl.when(j + INFLIGHT < NCH)
            def _():
                @pl.when(j >= 1)
                def _():
                    wait_out(j - 1)

                start_gather(j + INFLIGHT)

            return carry

        lax.fori_loop(0, NCH, step, 0)
        for j in range(NCH - NBUF, NCH):
            wait_out(j)

    return body(xg, table)


def kernel(x, table):
    xg = x.astype(jnp.int32).reshape(W, NCH, CHUNK)
    out = _embedding_lookup(xg, table)
    return out.reshape(NB, S, DIM)


# speedup vs baseline: 1.0014x; 1.0014x over previous
import functools

import jax
import jax.numpy as jnp
from jax import lax
from jax.experimental import pallas as pl
from jax.experimental.pallas import tpu as pltpu
from jax.experimental.pallas import tpu_sc as plsc

S = 50
NB = 16384
DIM = 64
W = 32
CHUNK = 128
ROWS = NB * S
RPW = ROWS // W
NCH = RPW // CHUNK
NBUF = 6
INFLIGHT = 5


def _embedding_lookup(xg, table):
    mesh = plsc.VectorSubcoreMesh(core_axis_name="c", subcore_axis_name="s")

    @functools.partial(
        pl.kernel,
        out_type=jax.ShapeDtypeStruct((ROWS // CHUNK, CHUNK, DIM),
                                      jnp.float32),
        mesh=mesh,
        scratch_types=[
            pltpu.VMEM((NCH, CHUNK), jnp.int32),
            pltpu.VMEM((NBUF, CHUNK, DIM), jnp.float32),
            pltpu.SemaphoreType.DMA((NBUF,)),
            pltpu.SemaphoreType.DMA((NBUF,)),
        ],
        compiler_params=pltpu.CompilerParams(use_tc_tiling_on_sc=False),
    )
    def body(xg_hbm, table_hbm, out_hbm, idx_v, rows_v, g_sem, o_sem):
        wid = lax.axis_index("s") * 2 + lax.axis_index("c")
        ch_base = wid * NCH
        pltpu.sync_copy(xg_hbm.at[wid], idx_v)

        def start_gather(j):
            buf = lax.rem(j, NBUF)
            pltpu.async_copy(table_hbm.at[idx_v.at[j]], rows_v.at[buf],
                             g_sem.at[buf])

        def wait_gather(j):
            buf = lax.rem(j, NBUF)
            pltpu.make_async_copy(table_hbm.at[idx_v.at[j]], rows_v.at[buf],
                                  g_sem.at[buf]).wait()

        def start_out(j):
            buf = lax.rem(j, NBUF)
            pltpu.async_copy(rows_v.at[buf], out_hbm.at[ch_base + j],
                             o_sem.at[buf])

        def wait_out(j):
            buf = lax.rem(j, NBUF)
            pltpu.make_async_copy(rows_v.at[buf], out_hbm.at[ch_base + j],
                                  o_sem.at[buf]).wait()

        for j in range(INFLIGHT):
            start_gather(j)

        def step(j, carry):
            wait_gather(j)
            start_out(j)

            @pl.when(j + INFLIGHT < NCH)
            def _():
                @pl.when(j >= 1)
                def _():
                    wait_out(j - 1)

                start_gather(j + INFLIGHT)

            return carry

        lax.fori_loop(0, NCH, step, 0)
        for j in range(NCH - NBUF, NCH):
            wait_out(j)

    return body(xg, table)


def kernel(x, table):
    xg = x.astype(jnp.int32).reshape(W, NCH, CHUNK)
    out = _embedding_lookup(xg, table)
    return out.reshape(NB, S, DIM)
